# baseline (speedup 1.0000x reference)
import jax
import jax.numpy as jnp
from jax import lax
from jax.experimental import pallas as pl
from jax.experimental.pallas import tpu as pltpu
from jax.experimental.pallas import tpu_sc as plsc

_VOCAB = 1000000
_HIDDEN = 64
_BATCH = 16384
_HIST = 200

_NC = 2
_NS = 16
_NW = _NC * _NS
_RPW = _BATCH // _NW


def _body(x_hbm, w_hbm, out_hbm,
          idx0, idx1, rows0, rows1, sg0, sg1, sw0, sw1):
    wid = lax.axis_index("s") * _NC + lax.axis_index("c")
    base = wid * _RPW

    def start_row(idx_v, rows_v, sg, b):
        pltpu.sync_copy(x_hbm.at[b], idx_v)
        pltpu.async_copy(w_hbm.at[idx_v], rows_v, sg)

    def wait_gather(idx_v, rows_v, sg):
        pltpu.make_async_copy(w_hbm.at[idx_v], rows_v, sg).wait()

    def start_write(rows_v, sw, b):
        pltpu.async_copy(rows_v, out_hbm.at[b], sw)

    def wait_write(rows_v, sw, b):
        pltpu.make_async_copy(rows_v, out_hbm.at[b], sw).wait()

    start_row(idx0, rows0, sg0, base)
    start_row(idx1, rows1, sg1, base + 1)

    def step(j, carry):
        b0 = base + j * 2

        wait_gather(idx0, rows0, sg0)
        start_write(rows0, sw0, b0)

        @pl.when(j * 2 + 2 < _RPW)
        def _():
            wait_write(rows0, sw0, b0)
            start_row(idx0, rows0, sg0, b0 + 2)

        wait_gather(idx1, rows1, sg1)
        start_write(rows1, sw1, b0 + 1)

        @pl.when(j * 2 + 3 < _RPW)
        def _():
            wait_write(rows1, sw1, b0 + 1)
            start_row(idx1, rows1, sg1, b0 + 3)

        return carry

    lax.fori_loop(0, _RPW // 2, step, 0)

    wait_write(rows0, sw0, base + _RPW - 2)
    wait_write(rows1, sw1, base + _RPW - 1)


def kernel(x, weight):
    mesh = plsc.VectorSubcoreMesh(
        core_axis_name="c", subcore_axis_name="s",
        num_cores=_NC, num_subcores=_NS)
    out = pl.kernel(
        _body,
        out_type=jax.ShapeDtypeStruct((_BATCH, _HIST, _HIDDEN), jnp.float32),
        mesh=mesh,
        compiler_params=pltpu.CompilerParams(use_tc_tiling_on_sc=False),
        scratch_types=[
            pltpu.VMEM((_HIST,), jnp.int32),
            pltpu.VMEM((_HIST,), jnp.int32),
            pltpu.VMEM((_HIST, _HIDDEN), jnp.float32),
            pltpu.VMEM((_HIST, _HIDDEN), jnp.float32),
            pltpu.SemaphoreType.DMA,
            pltpu.SemaphoreType.DMA,
            pltpu.SemaphoreType.DMA,
            pltpu.SemaphoreType.DMA,
        ],
    )(x.astype(jnp.int32), weight)
    return out
